# baseline (speedup 1.0000x reference)
import jax
import jax.numpy as jnp
from jax import lax
from jax.experimental import pallas as pl
from jax.experimental.pallas import tpu as pltpu
from jax.experimental.pallas import tpu_sc as plsc

_NQ = 256
_B = 8
_T = 8192
_BSC = 4
_NW = 32
_WPB = _NW // _BSC
_TW = _T // _WPB
_TB = 128
_NCHUNK = _TW // _TB


def _sc_body(idx_hbm, out_hbm, idx_v, buf0, buf1, sem0, sem1):
    nc = 2
    wid = lax.axis_index("s") * nc + lax.axis_index("c")
    b = wid // _WPB
    tbase = (wid % _WPB) * _TW

    pltpu.sync_copy(idx_hbm.at[b, pl.ds(tbase, _TW)], idx_v)

    zeros16 = jnp.zeros((16,), jnp.float32)
    ones16 = jnp.ones((16,), jnp.float32)
    iota16 = lax.iota(jnp.int32, 16)

    def _zbody(i, carry):
        r = i >> 3
        col = (i & 7) * 16
        buf0[r, pl.ds(col, 16)] = zeros16
        buf1[r, pl.ds(col, 16)] = zeros16
        return carry

    lax.fori_loop(0, (_NQ * _TB) // 16, _zbody, 0)

    def _scatter(buf, c, vals):
        for j in range(_TB // 16):
            v_idx = idx_v[pl.ds(c * _TB + j * 16, 16)]
            col = iota16 + (j * 16)
            plsc.store_scatter(buf, [v_idx, col], vals)

    bufs = (buf0, buf1)
    sems = (sem0, sem1)
    copies = [None, None]
    for c in range(_NCHUNK):
        k = c & 1
        buf = bufs[k]
        if c >= 2:
            copies[k].wait()
            _scatter(buf, c - 2, zeros16)
        _scatter(buf, c, ones16)
        cp = pltpu.make_async_copy(
            buf, out_hbm.at[b, :, pl.ds(tbase + c * _TB, _TB)], sems[k]
        )
        cp.start()
        copies[k] = cp
    copies[0].wait()
    copies[1].wait()


def _tc_body(idx_ref, out_ref):
    tb = out_ref.shape[2]
    iota = lax.broadcasted_iota(jnp.int32, (_NQ, tb), 0)
    out_ref[0] = (idx_ref[0] == iota).astype(jnp.float32)


def kernel(in_snd_slice, quant_onehot):
    idx = in_snd_slice.astype(jnp.int32)

    sc = pl.kernel(
        _sc_body,
        mesh=plsc.VectorSubcoreMesh(core_axis_name="c", subcore_axis_name="s"),
        out_type=jax.ShapeDtypeStruct((_BSC, _NQ, _T), jnp.float32),
        scratch_types=[
            pltpu.VMEM((_TW,), jnp.int32),
            pltpu.VMEM((_NQ, _TB), jnp.float32),
            pltpu.VMEM((_NQ, _TB), jnp.float32),
            pltpu.SemaphoreType.DMA,
            pltpu.SemaphoreType.DMA,
        ],
        compiler_params=pltpu.CompilerParams(needs_layout_passes=False),
    )
    out_sc = sc(idx[:_BSC])

    btc = _B - _BSC
    tb = 1024
    idx3 = idx[_BSC:].reshape(btc, 1, _T)
    out_tc = pl.pallas_call(
        _tc_body,
        grid=(btc, _T // tb),
        in_specs=[pl.BlockSpec((1, 1, tb), lambda b, t: (b, 0, t))],
        out_specs=pl.BlockSpec((1, _NQ, tb), lambda b, t: (b, 0, t)),
        out_shape=jax.ShapeDtypeStruct((btc, _NQ, _T), jnp.float32),
    )(idx3)

    return jnp.concatenate([out_sc, out_tc], axis=0)


# speedup vs baseline: 1.9352x; 1.9352x over previous
import jax
import jax.numpy as jnp
from jax import lax
from jax.experimental import pallas as pl
from jax.experimental.pallas import tpu as pltpu
from jax.experimental.pallas import tpu_sc as plsc

_NQ = 256
_B = 8
_T = 8192
_NW = 32
_WPB = _NW // _B
_TW = _T // _WPB
_TB = 128
_NCHUNK = _TW // _TB


def _sc_body(idx_hbm, out_hbm, idx_v, buf0, buf1, sem0, sem1, isem):
    nc = 2
    wid = lax.axis_index("s") * nc + lax.axis_index("c")
    b = wid // _WPB
    tbase = (wid % _WPB) * _TW

    idx_cp = pltpu.make_async_copy(idx_hbm.at[b, pl.ds(tbase, _TW)], idx_v, isem)
    idx_cp.start()

    zeros16 = jnp.zeros((16,), jnp.float32)
    ones16 = jnp.ones((16,), jnp.float32)
    iota16 = lax.iota(jnp.int32, 16)

    def _zbody(r, carry):
        for j in range(_TB // 16):
            buf0[r, pl.ds(j * 16, 16)] = zeros16
            buf1[r, pl.ds(j * 16, 16)] = zeros16
        return carry

    lax.fori_loop(0, _NQ, _zbody, 0)
    idx_cp.wait()

    def _scatter(buf, c, vals):
        for j in range(_TB // 16):
            v_idx = idx_v[pl.ds(c * _TB + j * 16, 16)]
            col = iota16 + (j * 16)
            plsc.store_scatter(buf, [v_idx, col], vals)

    bufs = (buf0, buf1)
    sems = (sem0, sem1)
    copies = [None, None]
    for c in range(_NCHUNK):
        k = c & 1
        buf = bufs[k]
        if c >= 2:
            copies[k].wait()
            _scatter(buf, c - 2, zeros16)
        _scatter(buf, c, ones16)
        cp = pltpu.make_async_copy(
            buf, out_hbm.at[b, :, pl.ds(tbase + c * _TB, _TB)], sems[k]
        )
        cp.start()
        copies[k] = cp
    copies[0].wait()
    copies[1].wait()


def kernel(in_snd_slice, quant_onehot):
    idx = in_snd_slice.astype(jnp.int32)
    sc = pl.kernel(
        _sc_body,
        mesh=plsc.VectorSubcoreMesh(core_axis_name="c", subcore_axis_name="s"),
        out_type=jax.ShapeDtypeStruct((_B, _NQ, _T), jnp.float32),
        scratch_types=[
            pltpu.VMEM((_TW,), jnp.int32),
            pltpu.VMEM((_NQ, _TB), jnp.float32),
            pltpu.VMEM((_NQ, _TB), jnp.float32),
            pltpu.SemaphoreType.DMA,
            pltpu.SemaphoreType.DMA,
            pltpu.SemaphoreType.DMA,
        ],
        compiler_params=pltpu.CompilerParams(needs_layout_passes=False),
    )
    return sc(idx)


# speedup vs baseline: 1.9924x; 1.0296x over previous
import jax
import jax.numpy as jnp
from jax import lax
from jax.experimental import pallas as pl
from jax.experimental.pallas import tpu as pltpu
from jax.experimental.pallas import tpu_sc as plsc

_NQ = 256
_B = 8
_T = 8192
_NW = 32
_WPB = _NW // _B
_TW = _T // _WPB
_TB = 128
_NCHUNK = _TW // _TB


_NBUF = 3


def _sc_body(idx_hbm, out_hbm, idx_v, buf0, buf1, buf2, sem0, sem1, sem2, isem):
    nc = 2
    wid = lax.axis_index("s") * nc + lax.axis_index("c")
    b = wid // _WPB
    tbase = (wid % _WPB) * _TW

    idx_cp = pltpu.make_async_copy(idx_hbm.at[b, pl.ds(tbase, _TW)], idx_v, isem)
    idx_cp.start()

    zeros16 = jnp.zeros((16,), jnp.float32)
    ones16 = jnp.ones((16,), jnp.float32)
    iota16 = lax.iota(jnp.int32, 16)

    bufs = (buf0, buf1, buf2)
    sems = (sem0, sem1, sem2)

    def _zero(buf):
        def _zbody(r, carry):
            for j in range(_TB // 16):
                buf[r, pl.ds(j * 16, 16)] = zeros16
            return carry

        lax.fori_loop(0, _NQ, _zbody, 0)

    def _scatter(buf, c, vals):
        for j in range(_TB // 16):
            v_idx = idx_v[pl.ds(c * _TB + j * 16, 16)]
            col = iota16 + (j * 16)
            plsc.store_scatter(buf, [v_idx, col], vals)

    copies = [None] * _NBUF
    for c in range(_NCHUNK):
        k = c % _NBUF
        buf = bufs[k]
        if c < _NBUF:
            _zero(buf)
            if c == 0:
                idx_cp.wait()
        else:
            copies[k].wait()
            _scatter(buf, c - _NBUF, zeros16)
        _scatter(buf, c, ones16)
        cp = pltpu.make_async_copy(
            buf, out_hbm.at[b, :, pl.ds(tbase + c * _TB, _TB)], sems[k]
        )
        cp.start()
        copies[k] = cp
    for cp in copies:
        cp.wait()


def kernel(in_snd_slice, quant_onehot):
    idx = in_snd_slice.astype(jnp.int32)
    sc = pl.kernel(
        _sc_body,
        mesh=plsc.VectorSubcoreMesh(core_axis_name="c", subcore_axis_name="s"),
        out_type=jax.ShapeDtypeStruct((_B, _NQ, _T), jnp.float32),
        scratch_types=[
            pltpu.VMEM((_TW,), jnp.int32),
            pltpu.VMEM((_NQ, _TB), jnp.float32),
            pltpu.VMEM((_NQ, _TB), jnp.float32),
            pltpu.VMEM((_NQ, _TB), jnp.float32),
            pltpu.SemaphoreType.DMA,
            pltpu.SemaphoreType.DMA,
            pltpu.SemaphoreType.DMA,
            pltpu.SemaphoreType.DMA,
        ],
        compiler_params=pltpu.CompilerParams(needs_layout_passes=False),
    )
    return sc(idx)
